# baseline (speedup 1.0000x reference)
import functools

import jax
import jax.numpy as jnp
from jax import lax
from jax.experimental import pallas as pl
from jax.experimental.pallas import tpu as pltpu
from jax.experimental.pallas import tpu_sc as plsc

_NUM_CORES = 2
_NUM_SUBCORES = 16
_NW = _NUM_CORES * _NUM_SUBCORES

_CHUNK = 128
_NBUF = 4


_TCB = 2048


def _transpose_pack(t_t):
    d2, v = t_t.shape
    grid = (v + _TCB - 1) // _TCB

    def body(x_ref, o_ref):
        r = lax.broadcasted_iota(jnp.int32, (d2, d2), 0)
        c = lax.broadcasted_iota(jnp.int32, (d2, d2), 1)
        ident = (r == c).astype(jnp.float32)
        xt = lax.dot_general(
            x_ref[...],
            ident,
            (((0,), (0,)), ((), ())),
            precision=lax.Precision.HIGHEST,
        )
        xt = xt.reshape(_TCB // 2, 2, d2)
        o_ref[:, 0:d2] = xt[:, 0, :]
        o_ref[:, d2:] = xt[:, 1, :]

    return pl.pallas_call(
        body,
        grid=(grid,),
        in_specs=[pl.BlockSpec((d2, _TCB), lambda i: (0, i))],
        out_specs=pl.BlockSpec((_TCB // 2, 2 * d2), lambda i: (i, 0)),
        out_shape=jax.ShapeDtypeStruct((v // 2, 2 * d2), jnp.float32),
    )(t_t)


@functools.partial(jax.jit, static_argnums=(2, 3))
def _embedding_lookup(idx, table, n_chunks, d):
    assert n_chunks % _NBUF == 0
    mesh = plsc.VectorSubcoreMesh(core_axis_name="c", subcore_axis_name="s")
    per_w = n_chunks * _CHUNK
    total = _NW * per_w

    @functools.partial(
        pl.kernel,
        out_type=jax.ShapeDtypeStruct((2 * total, d), jnp.float32),
        mesh=mesh,
        compiler_params=pltpu.CompilerParams(use_tc_tiling_on_sc=False),
        scratch_types=[
            pltpu.VMEM((n_chunks, _CHUNK), jnp.int32),
            pltpu.VMEM((_NBUF, _CHUNK), jnp.int32),
            pltpu.VMEM((_NBUF, _CHUNK, 64), jnp.float32),
            [pltpu.SemaphoreType.DMA] * _NBUF,
            [pltpu.SemaphoreType.DMA] * _NBUF,
        ],
    )
    def emb(idx_hbm, tab_hbm, out_hbm, idx_v, dst_v, rows_v, gsems, wsems):
        wid = lax.axis_index("s") * _NUM_CORES + lax.axis_index("c")
        base = wid * per_w
        pltpu.sync_copy(idx_hbm.at[wid], idx_v)
        iot = lax.iota(jnp.int32, 16)

        def prep(j, slot):
            qbase = 2 * (base + j * _CHUNK)
            for t in range(_CHUNK // 16):
                dst_v[slot, pl.ds(16 * t, 16)] = qbase + 2 * (16 * t + iot)

        def gather_start(j, slot):
            pltpu.async_copy(tab_hbm.at[idx_v.at[j]], rows_v.at[slot], gsems[slot])

        def gather_wait(j, slot):
            pltpu.make_async_copy(
                tab_hbm.at[idx_v.at[j]], rows_v.at[slot], gsems[slot]
            ).wait()

        def scatter_start(slot):
            pltpu.async_copy(
                rows_v.at[slot], out_hbm.at[dst_v.at[slot]], wsems[slot]
            )

        def scatter_wait(slot):
            pltpu.make_async_copy(
                rows_v.at[slot], out_hbm.at[dst_v.at[slot]], wsems[slot]
            ).wait()

        for b in range(_NBUF - 1):
            prep(b, b)
            gather_start(b, b)

        def group(g, _):
            j0 = g * _NBUF
            for b in range(_NBUF):
                j = j0 + b
                ahead = j + _NBUF - 1
                prev_slot = (b - 1) % _NBUF

                @pl.when(j > 0)
                def _():
                    scatter_wait(prev_slot)

                @pl.when(ahead < n_chunks)
                def _():
                    prep(ahead, prev_slot)
                    gather_start(ahead, prev_slot)

                gather_wait(j, b)
                scatter_start(b)
            return 0

        lax.fori_loop(0, n_chunks // _NBUF, group, 0, unroll=False)
        scatter_wait(_NBUF - 1)

    return emb(idx, table)


def kernel(x, table):
    b, s = x.shape
    v, d = table.shape
    total = b * s
    per_w = total // _NW
    n_chunks = per_w // _CHUNK
    idx = x.reshape(_NW, n_chunks, _CHUNK)
    table_lin = _transpose_pack(table.T).reshape(v, d)
    out2 = _embedding_lookup(idx, table_lin, n_chunks, d)
    return out2.reshape(total, 2 * d)[:, :d].reshape(b, s, d)


# speedup vs baseline: 1.2731x; 1.2731x over previous
import functools

import jax
import jax.numpy as jnp
from jax import lax
from jax.experimental import pallas as pl
from jax.experimental.pallas import tpu as pltpu
from jax.experimental.pallas import tpu_sc as plsc

_NUM_CORES = 2
_NUM_SUBCORES = 16
_NW = _NUM_CORES * _NUM_SUBCORES

_CHUNK = 128
_NBUF = 4


_TCB = 2048


def _transpose_pack(t_t):
    d2, v = t_t.shape
    grid = (v + _TCB - 1) // _TCB

    def body(x_ref, o_ref):
        xt = x_ref[...].T.reshape(_TCB // 2, 2, d2)
        o_ref[:, 0:d2] = xt[:, 0, :]
        o_ref[:, d2:] = xt[:, 1, :]

    return pl.pallas_call(
        body,
        grid=(grid,),
        in_specs=[pl.BlockSpec((d2, _TCB), lambda i: (0, i))],
        out_specs=pl.BlockSpec((_TCB // 2, 2 * d2), lambda i: (i, 0)),
        out_shape=jax.ShapeDtypeStruct((v // 2, 2 * d2), jnp.float32),
    )(t_t)


@functools.partial(jax.jit, static_argnums=(2, 3))
def _embedding_lookup(idx, table, n_chunks, d):
    assert n_chunks % _NBUF == 0
    mesh = plsc.VectorSubcoreMesh(core_axis_name="c", subcore_axis_name="s")
    per_w = n_chunks * _CHUNK
    total = _NW * per_w

    @functools.partial(
        pl.kernel,
        out_type=jax.ShapeDtypeStruct((2 * total, d), jnp.float32),
        mesh=mesh,
        compiler_params=pltpu.CompilerParams(use_tc_tiling_on_sc=False),
        scratch_types=[
            pltpu.VMEM((n_chunks, _CHUNK), jnp.int32),
            pltpu.VMEM((_NBUF, _CHUNK), jnp.int32),
            pltpu.VMEM((_NBUF, _CHUNK, 64), jnp.float32),
            [pltpu.SemaphoreType.DMA] * _NBUF,
            [pltpu.SemaphoreType.DMA] * _NBUF,
        ],
    )
    def emb(idx_hbm, tab_hbm, out_hbm, idx_v, dst_v, rows_v, gsems, wsems):
        wid = lax.axis_index("s") * _NUM_CORES + lax.axis_index("c")
        base = wid * per_w
        pltpu.sync_copy(idx_hbm.at[wid], idx_v)
        iot = lax.iota(jnp.int32, 16)

        def prep(j, slot):
            qbase = 2 * (base + j * _CHUNK)
            for t in range(_CHUNK // 16):
                dst_v[slot, pl.ds(16 * t, 16)] = qbase + 2 * (16 * t + iot)

        def gather_start(j, slot):
            pltpu.async_copy(tab_hbm.at[idx_v.at[j]], rows_v.at[slot], gsems[slot])

        def gather_wait(j, slot):
            pltpu.make_async_copy(
                tab_hbm.at[idx_v.at[j]], rows_v.at[slot], gsems[slot]
            ).wait()

        def scatter_start(slot):
            pltpu.async_copy(
                rows_v.at[slot], out_hbm.at[dst_v.at[slot]], wsems[slot]
            )

        def scatter_wait(slot):
            pltpu.make_async_copy(
                rows_v.at[slot], out_hbm.at[dst_v.at[slot]], wsems[slot]
            ).wait()

        for b in range(_NBUF - 1):
            prep(b, b)
            gather_start(b, b)

        def group(g, _):
            j0 = g * _NBUF
            for b in range(_NBUF):
                j = j0 + b
                ahead = j + _NBUF - 1
                prev_slot = (b - 1) % _NBUF

                @pl.when(j > 0)
                def _():
                    scatter_wait(prev_slot)

                @pl.when(ahead < n_chunks)
                def _():
                    prep(ahead, prev_slot)
                    gather_start(ahead, prev_slot)

                gather_wait(j, b)
                scatter_start(b)
            return 0

        lax.fori_loop(0, n_chunks // _NBUF, group, 0, unroll=False)
        scatter_wait(_NBUF - 1)

    return emb(idx, table)


def kernel(x, table):
    b, s = x.shape
    v, d = table.shape
    total = b * s
    per_w = total // _NW
    n_chunks = per_w // _CHUNK
    idx = x.reshape(_NW, n_chunks, _CHUNK)
    table_lin = _transpose_pack(table.T).reshape(v, d)
    out2 = _embedding_lookup(idx, table_lin, n_chunks, d)
    return out2.reshape(total, 2 * d)[:, :d].reshape(b, s, d)


# speedup vs baseline: 1.4679x; 1.1530x over previous
import functools

import jax
import jax.numpy as jnp
from jax import lax
from jax.experimental import pallas as pl
from jax.experimental.pallas import tpu as pltpu
from jax.experimental.pallas import tpu_sc as plsc

_NUM_CORES = 2
_NUM_SUBCORES = 16
_NW = _NUM_CORES * _NUM_SUBCORES

_CHUNK = 128
_NBUF = 4


_TCB = 4096


def _transpose_pack(t_t):
    d2, v = t_t.shape
    grid = (v + _TCB - 1) // _TCB

    def body(x_ref, o_ref):
        xt = x_ref[...].T.reshape(_TCB // 2, 2, d2)
        o_ref[:, 0:d2] = xt[:, 0, :]
        o_ref[:, d2:] = xt[:, 1, :]

    return pl.pallas_call(
        body,
        grid=(grid,),
        in_specs=[pl.BlockSpec((d2, _TCB), lambda i: (0, i))],
        out_specs=pl.BlockSpec((_TCB // 2, 2 * d2), lambda i: (i, 0)),
        out_shape=jax.ShapeDtypeStruct((v // 2, 2 * d2), jnp.float32),
    )(t_t)


@functools.partial(jax.jit, static_argnums=(2, 3))
def _embedding_lookup(idx, table, n_chunks, d):
    assert n_chunks % _NBUF == 0
    mesh = plsc.VectorSubcoreMesh(core_axis_name="c", subcore_axis_name="s")
    per_w = n_chunks * _CHUNK
    total = _NW * per_w

    @functools.partial(
        pl.kernel,
        out_type=jax.ShapeDtypeStruct((2 * total, d), jnp.float32),
        mesh=mesh,
        compiler_params=pltpu.CompilerParams(use_tc_tiling_on_sc=False),
        scratch_types=[
            pltpu.VMEM((n_chunks, _CHUNK), jnp.int32),
            pltpu.VMEM((_NBUF, _CHUNK), jnp.int32),
            pltpu.VMEM((_NBUF, _CHUNK, 64), jnp.float32),
            [pltpu.SemaphoreType.DMA] * _NBUF,
            [pltpu.SemaphoreType.DMA] * _NBUF,
        ],
    )
    def emb(idx_hbm, tab_hbm, out_hbm, idx_v, dst_v, rows_v, gsems, wsems):
        wid = lax.axis_index("s") * _NUM_CORES + lax.axis_index("c")
        base = wid * per_w
        pltpu.sync_copy(idx_hbm.at[wid], idx_v)
        iot = lax.iota(jnp.int32, 16)

        def prep(j, slot):
            qbase = 2 * (base + j * _CHUNK)
            for t in range(_CHUNK // 16):
                dst_v[slot, pl.ds(16 * t, 16)] = qbase + 2 * (16 * t + iot)

        def gather_start(j, slot):
            pltpu.async_copy(tab_hbm.at[idx_v.at[j]], rows_v.at[slot], gsems[slot])

        def gather_wait(j, slot):
            pltpu.make_async_copy(
                tab_hbm.at[idx_v.at[j]], rows_v.at[slot], gsems[slot]
            ).wait()

        def scatter_start(slot):
            pltpu.async_copy(
                rows_v.at[slot], out_hbm.at[dst_v.at[slot]], wsems[slot]
            )

        def scatter_wait(slot):
            pltpu.make_async_copy(
                rows_v.at[slot], out_hbm.at[dst_v.at[slot]], wsems[slot]
            ).wait()

        for b in range(_NBUF - 1):
            prep(b, b)
            gather_start(b, b)

        def group(g, _):
            j0 = g * _NBUF
            for b in range(_NBUF):
                j = j0 + b
                ahead = j + _NBUF - 1
                prev_slot = (b - 1) % _NBUF

                @pl.when(j > 0)
                def _():
                    scatter_wait(prev_slot)

                @pl.when(ahead < n_chunks)
                def _():
                    prep(ahead, prev_slot)
                    gather_start(ahead, prev_slot)

                gather_wait(j, b)
                scatter_start(b)
            return 0

        lax.fori_loop(0, n_chunks // _NBUF, group, 0, unroll=False)
        scatter_wait(_NBUF - 1)

    return emb(idx, table)


def kernel(x, table):
    b, s = x.shape
    v, d = table.shape
    total = b * s
    per_w = total // _NW
    n_chunks = per_w // _CHUNK
    idx = x.reshape(_NW, n_chunks, _CHUNK)
    table_lin = _transpose_pack(table.T).reshape(v, d)
    out2 = _embedding_lookup(idx, table_lin, n_chunks, d)
    return out2.reshape(total, 2 * d)[:, :d].reshape(b, s, d)


# speedup vs baseline: 1.5385x; 1.0481x over previous
import functools

import jax
import jax.numpy as jnp
from jax import lax
from jax.experimental import pallas as pl
from jax.experimental.pallas import tpu as pltpu
from jax.experimental.pallas import tpu_sc as plsc

_NUM_CORES = 2
_NUM_SUBCORES = 16
_NW = _NUM_CORES * _NUM_SUBCORES

_CHUNK = 128
_NBUF = 4


_TCB = 8192


def _transpose_pack(t_t):
    d2, v = t_t.shape
    grid = (v + _TCB - 1) // _TCB

    def body(x_ref, o_ref):
        xt = x_ref[...].T.reshape(_TCB // 2, 2, d2)
        o_ref[:, 0:d2] = xt[:, 0, :]
        o_ref[:, d2:] = xt[:, 1, :]

    return pl.pallas_call(
        body,
        grid=(grid,),
        in_specs=[pl.BlockSpec((d2, _TCB), lambda i: (0, i))],
        out_specs=pl.BlockSpec((_TCB // 2, 2 * d2), lambda i: (i, 0)),
        out_shape=jax.ShapeDtypeStruct((v // 2, 2 * d2), jnp.float32),
    )(t_t)


@functools.partial(jax.jit, static_argnums=(2, 3))
def _embedding_lookup(idx, table, n_chunks, d):
    assert n_chunks % _NBUF == 0
    mesh = plsc.VectorSubcoreMesh(core_axis_name="c", subcore_axis_name="s")
    per_w = n_chunks * _CHUNK
    total = _NW * per_w

    @functools.partial(
        pl.kernel,
        out_type=jax.ShapeDtypeStruct((2 * total, d), jnp.float32),
        mesh=mesh,
        compiler_params=pltpu.CompilerParams(use_tc_tiling_on_sc=False),
        scratch_types=[
            pltpu.VMEM((n_chunks, _CHUNK), jnp.int32),
            pltpu.VMEM((_NBUF, _CHUNK), jnp.int32),
            pltpu.VMEM((_NBUF, _CHUNK, 64), jnp.float32),
            [pltpu.SemaphoreType.DMA] * _NBUF,
            [pltpu.SemaphoreType.DMA] * _NBUF,
        ],
    )
    def emb(idx_hbm, tab_hbm, out_hbm, idx_v, dst_v, rows_v, gsems, wsems):
        wid = lax.axis_index("s") * _NUM_CORES + lax.axis_index("c")
        base = wid * per_w
        pltpu.sync_copy(idx_hbm.at[wid], idx_v)
        iot = lax.iota(jnp.int32, 16)

        def prep(j, slot):
            qbase = 2 * (base + j * _CHUNK)
            for t in range(_CHUNK // 16):
                dst_v[slot, pl.ds(16 * t, 16)] = qbase + 2 * (16 * t + iot)

        def gather_start(j, slot):
            pltpu.async_copy(tab_hbm.at[idx_v.at[j]], rows_v.at[slot], gsems[slot])

        def gather_wait(j, slot):
            pltpu.make_async_copy(
                tab_hbm.at[idx_v.at[j]], rows_v.at[slot], gsems[slot]
            ).wait()

        def scatter_start(slot):
            pltpu.async_copy(
                rows_v.at[slot], out_hbm.at[dst_v.at[slot]], wsems[slot]
            )

        def scatter_wait(slot):
            pltpu.make_async_copy(
                rows_v.at[slot], out_hbm.at[dst_v.at[slot]], wsems[slot]
            ).wait()

        for b in range(_NBUF - 1):
            prep(b, b)
            gather_start(b, b)

        def group(g, _):
            j0 = g * _NBUF
            for b in range(_NBUF):
                j = j0 + b
                ahead = j + _NBUF - 1
                prev_slot = (b - 1) % _NBUF

                @pl.when(j > 0)
                def _():
                    scatter_wait(prev_slot)

                @pl.when(ahead < n_chunks)
                def _():
                    prep(ahead, prev_slot)
                    gather_start(ahead, prev_slot)

                gather_wait(j, b)
                scatter_start(b)
            return 0

        lax.fori_loop(0, n_chunks // _NBUF, group, 0, unroll=False)
        scatter_wait(_NBUF - 1)

    return emb(idx, table)


def kernel(x, table):
    b, s = x.shape
    v, d = table.shape
    total = b * s
    per_w = total // _NW
    n_chunks = per_w // _CHUNK
    idx = x.reshape(_NW, n_chunks, _CHUNK)
    table_lin = _transpose_pack(table.T).reshape(v, d)
    out2 = _embedding_lookup(idx, table_lin, n_chunks, d)
    return out2.reshape(total, 2 * d)[:, :d].reshape(b, s, d)


# speedup vs baseline: 1.5528x; 1.0093x over previous
import functools

import jax
import jax.numpy as jnp
from jax import lax
from jax.experimental import pallas as pl
from jax.experimental.pallas import tpu as pltpu
from jax.experimental.pallas import tpu_sc as plsc

_NUM_CORES = 2
_NUM_SUBCORES = 16
_NW = _NUM_CORES * _NUM_SUBCORES

_CHUNK = 128
_NBUF = 4


_TCB = 16384


def _transpose_pack(t_t):
    d2, v = t_t.shape
    grid = (v + _TCB - 1) // _TCB

    def body(x_ref, o_ref):
        xt = x_ref[...].T.reshape(_TCB // 2, 2, d2)
        o_ref[:, 0:d2] = xt[:, 0, :]
        o_ref[:, d2:] = xt[:, 1, :]

    return pl.pallas_call(
        body,
        grid=(grid,),
        in_specs=[pl.BlockSpec((d2, _TCB), lambda i: (0, i))],
        out_specs=pl.BlockSpec((_TCB // 2, 2 * d2), lambda i: (i, 0)),
        out_shape=jax.ShapeDtypeStruct((v // 2, 2 * d2), jnp.float32),
    )(t_t)


@functools.partial(jax.jit, static_argnums=(2, 3))
def _embedding_lookup(idx, table, n_chunks, d):
    assert n_chunks % _NBUF == 0
    mesh = plsc.VectorSubcoreMesh(core_axis_name="c", subcore_axis_name="s")
    per_w = n_chunks * _CHUNK
    total = _NW * per_w

    @functools.partial(
        pl.kernel,
        out_type=jax.ShapeDtypeStruct((2 * total, d), jnp.float32),
        mesh=mesh,
        compiler_params=pltpu.CompilerParams(use_tc_tiling_on_sc=False),
        scratch_types=[
            pltpu.VMEM((n_chunks, _CHUNK), jnp.int32),
            pltpu.VMEM((_NBUF, _CHUNK), jnp.int32),
            pltpu.VMEM((_NBUF, _CHUNK, 64), jnp.float32),
            [pltpu.SemaphoreType.DMA] * _NBUF,
            [pltpu.SemaphoreType.DMA] * _NBUF,
        ],
    )
    def emb(idx_hbm, tab_hbm, out_hbm, idx_v, dst_v, rows_v, gsems, wsems):
        wid = lax.axis_index("s") * _NUM_CORES + lax.axis_index("c")
        base = wid * per_w
        pltpu.sync_copy(idx_hbm.at[wid], idx_v)
        iot = lax.iota(jnp.int32, 16)

        def prep(j, slot):
            qbase = 2 * (base + j * _CHUNK)
            for t in range(_CHUNK // 16):
                dst_v[slot, pl.ds(16 * t, 16)] = qbase + 2 * (16 * t + iot)

        def gather_start(j, slot):
            pltpu.async_copy(tab_hbm.at[idx_v.at[j]], rows_v.at[slot], gsems[slot])

        def gather_wait(j, slot):
            pltpu.make_async_copy(
                tab_hbm.at[idx_v.at[j]], rows_v.at[slot], gsems[slot]
            ).wait()

        def scatter_start(slot):
            pltpu.async_copy(
                rows_v.at[slot], out_hbm.at[dst_v.at[slot]], wsems[slot]
            )

        def scatter_wait(slot):
            pltpu.make_async_copy(
                rows_v.at[slot], out_hbm.at[dst_v.at[slot]], wsems[slot]
            ).wait()

        for b in range(_NBUF - 1):
            prep(b, b)
            gather_start(b, b)

        def group(g, _):
            j0 = g * _NBUF
            for b in range(_NBUF):
                j = j0 + b
                ahead = j + _NBUF - 1
                prev_slot = (b - 1) % _NBUF

                @pl.when(j > 0)
                def _():
                    scatter_wait(prev_slot)

                @pl.when(ahead < n_chunks)
                def _():
                    prep(ahead, prev_slot)
                    gather_start(ahead, prev_slot)

                gather_wait(j, b)
                scatter_start(b)
            return 0

        lax.fori_loop(0, n_chunks // _NBUF, group, 0, unroll=False)
        scatter_wait(_NBUF - 1)

    return emb(idx, table)


def kernel(x, table):
    b, s = x.shape
    v, d = table.shape
    total = b * s
    per_w = total // _NW
    n_chunks = per_w // _CHUNK
    idx = x.reshape(_NW, n_chunks, _CHUNK)
    table_lin = _transpose_pack(table.T).reshape(v, d)
    out2 = _embedding_lookup(idx, table_lin, n_chunks, d)
    return out2.reshape(total, 2 * d)[:, :d].reshape(b, s, d)
